# baseline (speedup 1.0000x reference)
import functools

import jax
import jax.numpy as jnp
from jax.experimental import pallas as pl
from jax.experimental.pallas import tpu as pltpu

TOP_P = 0.9
_ROWS_PER_BLOCK = 8
_BISECT_ITERS = 32


_CHUNK = 12544
_KEY_NEG_INF = -2139095041


def _f32_key(bits):
    return jnp.where(bits < 0, bits ^ jnp.int32(0x7FFFFFFF), bits)


def _key_f32(key):
    bits = jnp.where(key < 0, key ^ jnp.int32(0x7FFFFFFF), key)
    return jax.lax.bitcast_convert_type(bits, jnp.float32)


def _chunk_slices(v):
    n_full = (v - 1) // _CHUNK
    bounds = [(c * _CHUNK, _CHUNK) for c in range(n_full)]
    bounds.append((n_full * _CHUNK, v - n_full * _CHUNK))
    return bounds


def _masked_mass(x_ref, e_ref, tau, v):
    parts = [
        jnp.sum(
            jnp.where(x_ref[:, b:b + w] > tau, e_ref[:, b:b + w],
                      jnp.float32(0.0)),
            axis=1, keepdims=True)
        for b, w in _chunk_slices(v)
    ]
    while len(parts) > 1:
        parts = [a + b for a, b in zip(parts[::2], parts[1::2])] + (
            [parts[-1]] if len(parts) % 2 else [])
    return parts[0]


def _nucleus_block(x_ref, filt_ref, prob_ref, tok_ref, e_ref):
    v = x_ref.shape[1]
    x = x_ref[...]
    m = jnp.max(x, axis=1, keepdims=True)
    mn = jnp.min(x, axis=1, keepdims=True)
    e_ref[...] = jnp.exp(x - m)
    z = jnp.sum(e_ref[...], axis=1, keepdims=True)
    p = jnp.float32(TOP_P) * z
    tok = jnp.argmax(x, axis=1).astype(jnp.int32)

    hi0 = _f32_key(jax.lax.bitcast_convert_type(m, jnp.int32))
    lo0 = jnp.maximum(
        _f32_key(jax.lax.bitcast_convert_type(mn, jnp.int32)) - 1,
        jnp.int32(_KEY_NEG_INF))

    def step(_, carry):
        lo, hi = carry
        mid = (lo & hi) + ((lo ^ hi) >> 1)
        mass = _masked_mass(x_ref, e_ref, _key_f32(mid), v)
        above = mass <= p
        return jnp.where(above, lo, mid + 1), jnp.where(above, mid, hi)

    _, thr = jax.lax.fori_loop(0, _BISECT_ITERS, step, (lo0, hi0))
    filt_ref[...] = jnp.where(x >= _key_f32(thr), x, jnp.float32(-jnp.inf))
    prob_ref[...] = jnp.broadcast_to(m, prob_ref.shape)
    tok_ref[...] = jnp.broadcast_to(tok[:, None], tok_ref.shape)


def _tc_nucleus(logits):
    b, v = logits.shape
    r = _ROWS_PER_BLOCK
    grid = (b // r,)
    filt, prob, tok = pl.pallas_call(
        _nucleus_block,
        grid=grid,
        in_specs=[pl.BlockSpec((r, v), lambda i: (i, 0))],
        out_specs=[
            pl.BlockSpec((r, v), lambda i: (i, 0)),
            pl.BlockSpec((r, 128), lambda i: (i, 0)),
            pl.BlockSpec((r, 128), lambda i: (i, 0)),
        ],
        out_shape=[
            jax.ShapeDtypeStruct((b, v), jnp.float32),
            jax.ShapeDtypeStruct((b, 128), jnp.float32),
            jax.ShapeDtypeStruct((b, 128), jnp.int32),
        ],
        scratch_shapes=[
            pltpu.VMEM((r, v), jnp.float32),
        ],
    )(logits)
    return filt, prob[:, 0], tok[:, 0]


from jax import lax
from jax.experimental.pallas import tpu_sc as plsc

_SC_CHUNK = 10000
_NEG_INF = float("-inf")


def _key16(v):
    bits = jax.lax.bitcast_convert_type(v, jnp.int32)
    return jnp.where(bits < 0, bits ^ jnp.int32(0x7FFFFFFF), bits)


def _iota16():
    return jax.lax.iota(jnp.int32, 16)


def _scan_window(read_vec, a0, p):
    iota = _iota16()

    def body(i, carry):
        a, found, bin_, gab = carry
        j = 15 - i
        v = read_vec(j)
        rev = lax.rev(v, (0,))
        c = plsc.cumsum(rev)
        crossed = (a + c) > p
        cnt = jnp.sum(crossed.astype(jnp.int32))
        f_lane = 16 - cnt
        sel = iota == f_lane
        c_f = jnp.sum(jnp.where(sel, c, jnp.float32(0.0)))
        m_f = jnp.sum(jnp.where(sel, rev, jnp.float32(0.0)))
        take = (found == 0) & (cnt > 0)
        found2 = found | (cnt > 0).astype(jnp.int32)
        bin2 = jnp.where(take, 16 * j + (15 - f_lane), bin_)
        gab2 = jnp.where(take, a + c_f - m_f, gab)
        a2 = a + jnp.where(found2 > 0, jnp.float32(0.0), jnp.sum(v))
        return a2, found2, bin2, gab2

    init = (a0, jnp.int32(0), jnp.int32(0), a0)
    _, found, bin_, gab = lax.fori_loop(0, 16, body, init)
    return bin_, gab, found


def _make_sc_body(b, v):
  def _sc_body(x_hbm, filt_hbm, prob_hbm, tok_hbm,
               xbuf, obuf, hist, hcoarse, stage_f, stage_i):
    wid = lax.axis_index("s") * 2 + lax.axis_index("c")
    rows_per = b // 32
    nchunks = v // _SC_CHUNK
    nvec = _SC_CHUNK // 16
    iota = _iota16()

    def row_body(rl, _):
        row = wid * rows_per + rl
        rbase = row * v

        def p1_chunk(ci, carry):
            mx, mi = carry
            pltpu.sync_copy(
                x_hbm.at[pl.ds(rbase + ci * _SC_CHUNK, _SC_CHUNK)], xbuf)

            def p1_vec(i, c2):
                mx2, mi2 = c2
                vv = xbuf[pl.ds(16 * i, 16)]
                idx = ci * _SC_CHUNK + 16 * i + iota
                upd = vv > mx2
                return (jnp.where(upd, vv, mx2), jnp.where(upd, idx, mi2))

            return lax.fori_loop(0, nvec, p1_vec, (mx, mi))

        mx0 = jnp.full((16,), _NEG_INF, jnp.float32)
        mi0 = jnp.full((16,), jnp.int32(0x7FFFFFFF), jnp.int32)
        mx, mi = lax.fori_loop(0, nchunks, p1_chunk, (mx0, mi0))
        m = jnp.max(mx)
        tok = jnp.min(jnp.where(mx == m, mi, jnp.int32(0x7FFFFFFF)))

        def z16(i, _):
            hist[pl.ds(16 * i, 16)] = jnp.zeros((16,), jnp.float32)
            return 0
        lax.fori_loop(0, 4096, z16, 0)

        def zc(i, _):
            hcoarse[pl.ds(16 * i, 16)] = jnp.zeros((16,), jnp.float32)
            return 0
        lax.fori_loop(0, 16, zc, 0)

        def p2_chunk(ci, zacc):
            pltpu.sync_copy(
                x_hbm.at[pl.ds(rbase + ci * _SC_CHUNK, _SC_CHUNK)], xbuf)

            def p2_vec(i, z2):
                vv = xbuf[pl.ds(16 * i, 16)]
                ev = jnp.exp(vv - m)
                key = _key16(vv)
                b1 = (key >> 16) + 32768
                plsc.addupdate_scatter(hist, [b1], ev)
                plsc.addupdate_scatter(hcoarse, [b1 >> 8], ev)
                return z2 + ev

            return lax.fori_loop(0, nvec, p2_vec, zacc)

        zacc = lax.fori_loop(0, nchunks, p2_chunk, jnp.zeros((16,),
                                                            jnp.float32))
        p = jnp.float32(TOP_P) * jnp.sum(zacc)

        bc1, g1, _ = _scan_window(
            lambda j: hcoarse[pl.ds(16 * j, 16)], jnp.float32(0.0), p)
        bf1, g2, _ = _scan_window(
            lambda j: hist[pl.ds(256 * bc1 + 16 * j, 16)], g1, p)
        kb_star = (256 * bc1 + bf1) - 32768

        lax.fori_loop(0, 4096, z16, 0)
        lax.fori_loop(0, 16, zc, 0)

        def p3_chunk(ci, _):
            pltpu.sync_copy(
                x_hbm.at[pl.ds(rbase + ci * _SC_CHUNK, _SC_CHUNK)], xbuf)

            def p3_vec(i, __):
                vv = xbuf[pl.ds(16 * i, 16)]
                ev = jnp.exp(vv - m)
                key = _key16(vv)
                sel = (key >> 16) == kb_star
                lo16 = key & 0xFFFF
                plsc.addupdate_scatter(hist, [lo16], ev, mask=sel)
                plsc.addupdate_scatter(hcoarse, [lo16 >> 8], ev, mask=sel)
                return 0

            lax.fori_loop(0, nvec, p3_vec, 0)
            return 0

        lax.fori_loop(0, nchunks, p3_chunk, 0)

        bc2, g3, _ = _scan_window(
            lambda j: hcoarse[pl.ds(16 * j, 16)], g2, p)
        bf2, _, _ = _scan_window(
            lambda j: hist[pl.ds(256 * bc2 + 16 * j, 16)], g3, p)
        u_star = (kb_star << 16) | (256 * bc2 + bf2)

        def p4_chunk(ci, _):
            pltpu.sync_copy(
                x_hbm.at[pl.ds(rbase + ci * _SC_CHUNK, _SC_CHUNK)], xbuf)

            def p4_vec(i, __):
                vv = xbuf[pl.ds(16 * i, 16)]
                keep = _key16(vv) >= u_star
                obuf[pl.ds(16 * i, 16)] = jnp.where(keep, vv, _NEG_INF)
                return 0

            lax.fori_loop(0, nvec, p4_vec, 0)
            pltpu.sync_copy(
                obuf, filt_hbm.at[pl.ds(rbase + ci * _SC_CHUNK, _SC_CHUNK)])
            return 0

        lax.fori_loop(0, nchunks, p4_chunk, 0)

        stage_f[...] = jnp.where(iota == 0, m, jnp.float32(0.0))
        stage_i[...] = jnp.where(iota == 0, tok, jnp.int32(0))
        pltpu.sync_copy(stage_f, prob_hbm.at[pl.ds(row * 16, 16)])
        pltpu.sync_copy(stage_i, tok_hbm.at[pl.ds(row * 16, 16)])
        return 0

    lax.fori_loop(0, rows_per, row_body, 0)

  return _sc_body


def _sc_nucleus(logits):
    b, v = logits.shape
    mesh = plsc.VectorSubcoreMesh(core_axis_name="c", subcore_axis_name="s",
                                  num_cores=2, num_subcores=16)
    filt, prob, tok = pl.kernel(
        _make_sc_body(b, v),
        out_type=[
            jax.ShapeDtypeStruct((b * v,), jnp.float32),
            jax.ShapeDtypeStruct((b * 16,), jnp.float32),
            jax.ShapeDtypeStruct((b * 16,), jnp.int32),
        ],
        mesh=mesh,
        scratch_types=[
            pltpu.VMEM((_SC_CHUNK,), jnp.float32),
            pltpu.VMEM((_SC_CHUNK,), jnp.float32),
            pltpu.VMEM((65536,), jnp.float32),
            pltpu.VMEM((256,), jnp.float32),
            pltpu.VMEM((16,), jnp.float32),
            pltpu.VMEM((16,), jnp.int32),
        ],
        compiler_params=pltpu.CompilerParams(needs_layout_passes=False),
    )(logits.reshape(b * v))
    return (filt.reshape(b, v), prob.reshape(b, 16)[:, 0],
            tok.reshape(b, 16)[:, 0])


@jax.jit
def kernel(logits):
    return _sc_nucleus(logits)


# speedup vs baseline: 3.5772x; 3.5772x over previous
import functools

import jax
import jax.numpy as jnp
from jax.experimental import pallas as pl
from jax.experimental.pallas import tpu as pltpu

TOP_P = 0.9
_ROWS_PER_BLOCK = 8
_BISECT_ITERS = 32


_CHUNK = 12544
_KEY_NEG_INF = -2139095041


def _f32_key(bits):
    return jnp.where(bits < 0, bits ^ jnp.int32(0x7FFFFFFF), bits)


def _key_f32(key):
    bits = jnp.where(key < 0, key ^ jnp.int32(0x7FFFFFFF), key)
    return jax.lax.bitcast_convert_type(bits, jnp.float32)


def _chunk_slices(v):
    n_full = (v - 1) // _CHUNK
    bounds = [(c * _CHUNK, _CHUNK) for c in range(n_full)]
    bounds.append((n_full * _CHUNK, v - n_full * _CHUNK))
    return bounds


def _masked_mass(x_ref, e_ref, tau, v):
    parts = [
        jnp.sum(
            jnp.where(x_ref[:, b:b + w] > tau, e_ref[:, b:b + w],
                      jnp.float32(0.0)),
            axis=1, keepdims=True)
        for b, w in _chunk_slices(v)
    ]
    while len(parts) > 1:
        parts = [a + b for a, b in zip(parts[::2], parts[1::2])] + (
            [parts[-1]] if len(parts) % 2 else [])
    return parts[0]


def _nucleus_block(x_ref, filt_ref, prob_ref, tok_ref, e_ref):
    v = x_ref.shape[1]
    x = x_ref[...]
    m = jnp.max(x, axis=1, keepdims=True)
    mn = jnp.min(x, axis=1, keepdims=True)
    e_ref[...] = jnp.exp(x - m)
    z = jnp.sum(e_ref[...], axis=1, keepdims=True)
    p = jnp.float32(TOP_P) * z
    tok = jnp.argmax(x, axis=1).astype(jnp.int32)

    hi0 = _f32_key(jax.lax.bitcast_convert_type(m, jnp.int32))
    lo0 = jnp.maximum(
        _f32_key(jax.lax.bitcast_convert_type(mn, jnp.int32)) - 1,
        jnp.int32(_KEY_NEG_INF))

    def step(_, carry):
        lo, hi = carry
        mid = (lo & hi) + ((lo ^ hi) >> 1)
        mass = _masked_mass(x_ref, e_ref, _key_f32(mid), v)
        above = mass <= p
        return jnp.where(above, lo, mid + 1), jnp.where(above, mid, hi)

    _, thr = jax.lax.fori_loop(0, _BISECT_ITERS, step, (lo0, hi0))
    filt_ref[...] = jnp.where(x >= _key_f32(thr), x, jnp.float32(-jnp.inf))
    prob_ref[...] = jnp.broadcast_to(m, prob_ref.shape)
    tok_ref[...] = jnp.broadcast_to(tok[:, None], tok_ref.shape)


def _tc_nucleus(logits):
    b, v = logits.shape
    r = _ROWS_PER_BLOCK
    grid = (b // r,)
    filt, prob, tok = pl.pallas_call(
        _nucleus_block,
        grid=grid,
        in_specs=[pl.BlockSpec((r, v), lambda i: (i, 0))],
        out_specs=[
            pl.BlockSpec((r, v), lambda i: (i, 0)),
            pl.BlockSpec((r, 128), lambda i: (i, 0)),
            pl.BlockSpec((r, 128), lambda i: (i, 0)),
        ],
        out_shape=[
            jax.ShapeDtypeStruct((b, v), jnp.float32),
            jax.ShapeDtypeStruct((b, 128), jnp.float32),
            jax.ShapeDtypeStruct((b, 128), jnp.int32),
        ],
        scratch_shapes=[
            pltpu.VMEM((r, v), jnp.float32),
        ],
    )(logits)
    return filt, prob[:, 0], tok[:, 0]


from jax import lax
from jax.experimental.pallas import tpu_sc as plsc

_SC_CHUNK = 20000
_NEG_INF = float("-inf")


def _key16(v):
    bits = jax.lax.bitcast_convert_type(v, jnp.int32)
    return jnp.where(bits < 0, bits ^ jnp.int32(0x7FFFFFFF), bits)


def _iota16():
    return jax.lax.iota(jnp.int32, 16)


def _unrolled_loop(n_iters, unroll, fn, carry):
    def outer(o, c):
        for u in range(unroll):
            c = fn(o * unroll + u, c)
        return c
    return lax.fori_loop(0, n_iters // unroll, outer, carry)


def _scan_window(read_vec, a0, p):
    iota = _iota16()

    def body(i, carry):
        a, found, bin_, gab = carry
        j = 15 - i
        v = read_vec(j)
        rev = lax.rev(v, (0,))
        c = plsc.cumsum(rev)
        crossed = (a + c) > p
        cnt = jnp.sum(crossed.astype(jnp.int32))
        f_lane = 16 - cnt
        sel = iota == f_lane
        c_f = jnp.sum(jnp.where(sel, c, jnp.float32(0.0)))
        m_f = jnp.sum(jnp.where(sel, rev, jnp.float32(0.0)))
        take = (found == 0) & (cnt > 0)
        found2 = found | (cnt > 0).astype(jnp.int32)
        bin2 = jnp.where(take, 16 * j + (15 - f_lane), bin_)
        gab2 = jnp.where(take, a + c_f - m_f, gab)
        a2 = a + jnp.where(found2 > 0, jnp.float32(0.0), jnp.sum(v))
        return a2, found2, bin2, gab2

    init = (a0, jnp.int32(0), jnp.int32(0), a0)
    _, found, bin_, gab = lax.fori_loop(0, 16, body, init)
    return bin_, gab, found


def _make_sc_body(b, v):
  def _sc_body(x_hbm, filt_hbm, prob_hbm, tok_hbm,
               xbuf, obuf, hist, hcoarse, stage_f, stage_i):
    wid = lax.axis_index("s") * 2 + lax.axis_index("c")
    rows_per = b // 32
    nchunks = v // _SC_CHUNK
    nvec = _SC_CHUNK // 16
    iota = _iota16()

    def row_body(rl, _):
        row = wid * rows_per + rl
        rbase = row * v

        def p1_chunk(ci, carry):
            mx, mi = carry
            pltpu.sync_copy(
                x_hbm.at[pl.ds(rbase + ci * _SC_CHUNK, _SC_CHUNK)], xbuf)

            def p1_vec(i, c2):
                mx2, mi2 = c2
                vv = xbuf[pl.ds(16 * i, 16)]
                idx = ci * _SC_CHUNK + 16 * i + iota
                upd = vv > mx2
                return (jnp.where(upd, vv, mx2), jnp.where(upd, idx, mi2))

            return _unrolled_loop(nvec, 25, p1_vec, (mx, mi))

        mx0 = jnp.full((16,), _NEG_INF, jnp.float32)
        mi0 = jnp.full((16,), jnp.int32(0x7FFFFFFF), jnp.int32)
        mx, mi = lax.fori_loop(0, nchunks, p1_chunk, (mx0, mi0))
        m = jnp.max(mx)
        tok = jnp.min(jnp.where(mx == m, mi, jnp.int32(0x7FFFFFFF)))

        def z16(i, _):
            hist[pl.ds(16 * i, 16)] = jnp.zeros((16,), jnp.float32)
            return 0
        _unrolled_loop(4096, 8, z16, 0)

        def build_coarse():
            def cj(j, _):
                acc = jnp.zeros((16,), jnp.float32)
                for l in range(16):
                    a = jnp.zeros((16,), jnp.float32)
                    for t in range(16):
                        a = a + hist[pl.ds(4096 * j + 256 * l + 16 * t, 16)]
                    acc = jnp.where(iota == l, jnp.sum(a), acc)
                hcoarse[pl.ds(16 * j, 16)] = acc
                return 0
            lax.fori_loop(0, 16, cj, 0)

        def p2_chunk(ci, zacc):
            pltpu.sync_copy(
                x_hbm.at[pl.ds(rbase + ci * _SC_CHUNK, _SC_CHUNK)], xbuf)

            def p2_vec(i, z2):
                vv = xbuf[pl.ds(16 * i, 16)]
                ev = jnp.exp(vv - m)
                b1 = (_key16(vv) >> 16) + 32768
                plsc.addupdate_scatter(hist, [b1], ev)
                return z2 + ev

            return _unrolled_loop(nvec, 25, p2_vec, zacc)

        zacc = lax.fori_loop(0, nchunks, p2_chunk, jnp.zeros((16,),
                                                            jnp.float32))
        p = jnp.float32(TOP_P) * jnp.sum(zacc)

        build_coarse()
        bc1, g1, _ = _scan_window(
            lambda j: hcoarse[pl.ds(16 * j, 16)], jnp.float32(0.0), p)
        bf1, g2, _ = _scan_window(
            lambda j: hist[pl.ds(256 * bc1 + 16 * j, 16)], g1, p)
        kb_star = (256 * bc1 + bf1) - 32768

        _unrolled_loop(4096, 8, z16, 0)

        def p3_chunk(ci, _):
            pltpu.sync_copy(
                x_hbm.at[pl.ds(rbase + ci * _SC_CHUNK, _SC_CHUNK)], xbuf)

            def p3_vec(i, __):
                vv = xbuf[pl.ds(16 * i, 16)]
                ev = jnp.exp(vv - m)
                key = _key16(vv)
                sel = (key >> 16) == kb_star
                plsc.addupdate_scatter(hist, [key & 0xFFFF], ev, mask=sel)
                return 0

            _unrolled_loop(nvec, 25, p3_vec, 0)
            return 0

        lax.fori_loop(0, nchunks, p3_chunk, 0)

        build_coarse()
        bc2, g3, _ = _scan_window(
            lambda j: hcoarse[pl.ds(16 * j, 16)], g2, p)
        bf2, _, _ = _scan_window(
            lambda j: hist[pl.ds(256 * bc2 + 16 * j, 16)], g3, p)
        u_star = (kb_star << 16) | (256 * bc2 + bf2)

        def p4_chunk(ci, _):
            pltpu.sync_copy(
                x_hbm.at[pl.ds(rbase + ci * _SC_CHUNK, _SC_CHUNK)], xbuf)

            def p4_vec(i, __):
                vv = xbuf[pl.ds(16 * i, 16)]
                keep = _key16(vv) >= u_star
                obuf[pl.ds(16 * i, 16)] = jnp.where(keep, vv, _NEG_INF)
                return 0

            _unrolled_loop(nvec, 25, p4_vec, 0)
            pltpu.sync_copy(
                obuf, filt_hbm.at[pl.ds(rbase + ci * _SC_CHUNK, _SC_CHUNK)])
            return 0

        lax.fori_loop(0, nchunks, p4_chunk, 0)

        stage_f[...] = jnp.where(iota == 0, m, jnp.float32(0.0))
        stage_i[...] = jnp.where(iota == 0, tok, jnp.int32(0))
        pltpu.sync_copy(stage_f, prob_hbm.at[pl.ds(row * 16, 16)])
        pltpu.sync_copy(stage_i, tok_hbm.at[pl.ds(row * 16, 16)])
        return 0

    lax.fori_loop(0, rows_per, row_body, 0)

  return _sc_body


def _sc_nucleus(logits):
    b, v = logits.shape
    mesh = plsc.VectorSubcoreMesh(core_axis_name="c", subcore_axis_name="s",
                                  num_cores=2, num_subcores=16)
    filt, prob, tok = pl.kernel(
        _make_sc_body(b, v),
        out_type=[
            jax.ShapeDtypeStruct((b * v,), jnp.float32),
            jax.ShapeDtypeStruct((b * 16,), jnp.float32),
            jax.ShapeDtypeStruct((b * 16,), jnp.int32),
        ],
        mesh=mesh,
        scratch_types=[
            pltpu.VMEM((_SC_CHUNK,), jnp.float32),
            pltpu.VMEM((_SC_CHUNK,), jnp.float32),
            pltpu.VMEM((65536,), jnp.float32),
            pltpu.VMEM((256,), jnp.float32),
            pltpu.VMEM((16,), jnp.float32),
            pltpu.VMEM((16,), jnp.int32),
        ],
        compiler_params=pltpu.CompilerParams(needs_layout_passes=False),
    )(logits.reshape(b * v))
    return (filt.reshape(b, v), prob.reshape(b, 16)[:, 0],
            tok.reshape(b, 16)[:, 0])


_SC_ROWS = 32


@jax.jit
def kernel(logits):
    b = logits.shape[0]
    tc_rows = b - _SC_ROWS
    tf, tp, tt = _tc_nucleus(logits[:tc_rows])
    sf, sp, st = _sc_nucleus(logits[tc_rows:])
    return (jnp.concatenate([tf, sf], axis=0),
            jnp.concatenate([tp, sp], axis=0),
            jnp.concatenate([tt, st], axis=0))
